# baseline (speedup 1.0000x reference)
import dataclasses
import functools

import jax
import jax.numpy as jnp
from jax import lax
from jax.experimental import pallas as pl
from jax.experimental.pallas import tpu as pltpu
from jax.experimental.pallas import tpu_sc as plsc

NUM_CORES = 2
NUM_SUBCORES = 16
LANES = 16

CHUNK = 64
NBUF = 2
N_LOC = 5632


def _sc_aggregate(xfull, xinit, rowp, colp, counts, cap, half, d):
    rows_per_tile = N_LOC // NUM_SUBCORES
    chunk = CHUNK
    nbuf = NBUF
    mesh = plsc.VectorSubcoreMesh(core_axis_name="c", subcore_axis_name="s")
    cp = pltpu.CompilerParams()
    if "needs_layout_passes" in pltpu.CompilerParams.__dataclass_fields__:
        cp = dataclasses.replace(cp, needs_layout_passes=False)

    @functools.partial(
        pl.kernel,
        out_type=jax.ShapeDtypeStruct((2 * N_LOC, 2, LANES * 8), jnp.float32),
        mesh=mesh,
        compiler_params=cp,
        scratch_types=[
            pltpu.VMEM_SHARED((N_LOC, 2, LANES * 8), jnp.float32),
            pltpu.VMEM((nbuf, chunk), jnp.int32),
            pltpu.VMEM((nbuf, chunk), jnp.int32),
            pltpu.VMEM((nbuf, chunk, 2, LANES * 8), jnp.float32),
            pltpu.VMEM((LANES,), jnp.int32),
        ] + [pltpu.SemaphoreType.DMA] * (3 * nbuf),
    )
    def sc_agg(xf_hbm, xi_hbm, row_hbm, col_hbm, cnt_hbm, out_hbm,
               acc, dsts, srcs, buf, cnt, *sems):
        sem_g = sems[:nbuf]
        sem_s = sems[nbuf:2 * nbuf]
        sem_i = sems[2 * nbuf:]
        cid = lax.axis_index("c")
        sid = lax.axis_index("s")
        rbase = sid * rows_per_tile
        init_cp = pltpu.async_copy(
            xi_hbm.at[pl.ds(cid * N_LOC + rbase, rows_per_tile)],
            acc.at[pl.ds(rbase, rows_per_tile)], sem_s[0])
        pltpu.sync_copy(cnt_hbm.at[cid], cnt)
        count = jnp.max(cnt[...])
        cpt = (count + chunk * NUM_SUBCORES - 1) // (chunk * NUM_SUBCORES)
        cpt = ((cpt + nbuf - 1) // nbuf) * nbuf
        cpt = jnp.maximum(cpt, nbuf)
        rounds = cpt // nbuf
        ebase = cid * cap + sid * cpt * chunk
        base_row = cid * half

        init_cp.wait()
        plsc.subcore_barrier()

        def stage_idx(c, b):
            pltpu.async_copy(row_hbm.at[pl.ds(ebase + c * chunk, chunk)],
                             dsts.at[b], sem_i[b])
            pltpu.async_copy(col_hbm.at[pl.ds(ebase + c * chunk, chunk)],
                             srcs.at[b], sem_i[b])

        def wait_idx(c, b):
            pltpu.make_async_copy(
                row_hbm.at[pl.ds(ebase + c * chunk, chunk)],
                dsts.at[b], sem_i[b]).wait()
            pltpu.make_async_copy(
                col_hbm.at[pl.ds(ebase + c * chunk, chunk)],
                srcs.at[b], sem_i[b]).wait()

        def transform(b):
            for i in range(chunk // LANES):
                sl = pl.ds(i * LANES, LANES)
                r = dsts.at[b, sl][...]
                c_ = srcs.at[b, sl][...]
                dv = lax.iota(jnp.int32, LANES) + (half + i * LANES)
                dsts.at[b, sl][...] = jnp.where(r == c_, dv, r - base_row)

        def start_gather(b):
            pltpu.async_copy(xf_hbm.at[srcs.at[b]], buf.at[b], sem_g[b])

        def wait_gather(b):
            pltpu.make_async_copy(
                xf_hbm.at[srcs.at[b]], buf.at[b], sem_g[b]).wait()

        for b in range(nbuf):
            stage_idx(b, b)
            wait_idx(b, b)
            transform(b)
            start_gather(b)

        @pl.loop(0, rounds - 1)
        def _(q):
            c0 = q * nbuf
            for b in range(nbuf):
                c = c0 + b
                wait_gather(b)
                pltpu.async_copy(buf.at[b], acc.at[dsts.at[b]], sem_s[b],
                                 add=True)
                pltpu.make_async_copy(
                    buf.at[b], acc.at[dsts.at[b]], sem_s[b]).wait()
                stage_idx(c + nbuf, b)
                wait_idx(c + nbuf, b)
                transform(b)
                start_gather(b)

        for b in range(nbuf):
            wait_gather(b)
            pltpu.sync_copy(buf.at[b], acc.at[dsts.at[b]], add=True)

        plsc.subcore_barrier()
        pltpu.sync_copy(
            acc.at[pl.ds(rbase, rows_per_tile)],
            out_hbm.at[pl.ds(cid * N_LOC + rbase, rows_per_tile)],
        )

    return sc_agg(xfull, xinit, rowp, colp, counts)


def _tc_mlp(s, W1, b1, W2, b2, d):
    bm = 1024
    n_rows = s.shape[0]

    def body(s_ref, w1_ref, b1_ref, w2_ref, b2_ref, o_ref):
        h = jnp.maximum(
            jnp.dot(s_ref[...], w1_ref[...],
                    preferred_element_type=jnp.float32) + b1_ref[...], 0.0)
        o_ref[...] = jnp.dot(h, w2_ref[...],
                             preferred_element_type=jnp.float32) + b2_ref[...]

    return pl.pallas_call(
        body,
        grid=(n_rows // bm,),
        in_specs=[
            pl.BlockSpec((bm, d), lambda i: (i, 0)),
            pl.BlockSpec((d, d), lambda i: (0, 0)),
            pl.BlockSpec((1, d), lambda i: (0, 0)),
            pl.BlockSpec((d, d), lambda i: (0, 0)),
            pl.BlockSpec((1, d), lambda i: (0, 0)),
        ],
        out_specs=pl.BlockSpec((bm, d), lambda i: (i, 0)),
        out_shape=jax.ShapeDtypeStruct((n_rows, d), jnp.float32),
    )(s, W1, b1, W2, b2)


def kernel(x_in, edge_index, W1, b1, W2, b2):
    n, d = x_in.shape
    e = edge_index.shape[1]
    half = n // 2
    nhi = n - half

    cap_quant = NUM_SUBCORES * CHUNK * NBUF
    cap = ((e + cap_quant - 1) // cap_quant) * cap_quant
    row = edge_index[0]
    col = edge_index[1]
    is_right = row >= half
    s_i = is_right.astype(jnp.int32)
    n_right = jnp.sum(s_i)
    left_pos = jnp.cumsum(1 - s_i) - (1 - s_i)
    right_pos = jnp.cumsum(s_i) - s_i
    pos = jnp.where(is_right, cap + right_pos, left_pos)
    rowp = jnp.zeros((2 * cap,), jnp.int32).at[pos].set(row)
    colp = jnp.zeros((2 * cap,), jnp.int32).at[pos].set(col)
    counts = jnp.stack([
        jnp.full((LANES,), e, jnp.int32) - n_right.astype(jnp.int32),
        jnp.broadcast_to(n_right.astype(jnp.int32), (LANES,)),
    ])

    xinit = jnp.concatenate([
        jnp.pad(x_in[:half], ((0, N_LOC - half), (0, 0))),
        jnp.pad(x_in[half:], ((0, N_LOC - nhi), (0, 0))),
    ], axis=0)

    sums = _sc_aggregate(x_in.reshape(n, 2, 128), xinit.reshape(2 * N_LOC, 2, 128), rowp, colp, counts, cap, half, d).reshape(2 * N_LOC, d)
    out = _tc_mlp(sums, W1, b1[None, :], W2, b2[None, :], d)
    return jnp.concatenate([out[:half], out[N_LOC:N_LOC + nhi]], axis=0)
